# baseline (speedup 1.0000x reference)
import functools

import jax
import jax.numpy as jnp
import numpy as np
from jax.experimental import pallas as pl
from jax.experimental.pallas import tpu as pltpu

B = 1024
N = 64
T = 8
DH = 32
DO = 64
P = 8
F = T * DH
BB = 32

_M0 = np.zeros((2 * T, T), np.float32)
_M1 = np.zeros((2 * T, T), np.float32)
for _t in range(T):
    _M0[2 * _t, _t] += 1.0
    _M0[2 * T - 2, _t] += -1.0
    _M1[2 * _t + 1, _t] += 1.0
    _M1[2 * T - 1, _t] += -1.0


def _tc_body(xe_ref, xn_ref, We_ref, bt_ref, W1_ref, b1_ref,
             W2_ref, b2_ref, W3_ref, b3_ref, f_re_ref, mf_ref):
    xe = xe_ref[...]
    xn = xn_ref[...]
    We = We_ref[...]
    bt = bt_ref[...]

    f_ego = jnp.maximum(jnp.dot(xe, We, preferred_element_type=jnp.float32)
                        + bt, 0.0)
    f_nei = jnp.maximum(jnp.dot(xn, We, preferred_element_type=jnp.float32)
                        + bt, 0.0)

    f = (f_ego[:, None, :] * f_nei.reshape(BB, N, F)).reshape(BB * N, F)

    h = jnp.maximum(jnp.dot(f, W1_ref[...],
                            preferred_element_type=jnp.float32) + b1_ref[...],
                    0.0)
    h = jnp.maximum(jnp.dot(h, W2_ref[...],
                            preferred_element_type=jnp.float32) + b2_ref[...],
                    0.0)
    f_re_ref[...] = jnp.maximum(
        jnp.dot(h, W3_ref[...], preferred_element_type=jnp.float32)
        + b3_ref[...], 0.0)

    s = jnp.sum(xn, axis=1)
    mf_ref[...] = (s != 0.0).astype(jnp.float32)


def _tc_mlp(xe_flat, xn_flat, W_exp, b_tile, W1, b1, W2, b2, W3, b3,
            interpret=False):
    grid = (B // BB,)
    return pl.pallas_call(
        _tc_body,
        grid=grid,
        in_specs=[
            pl.BlockSpec((BB, 2 * T), lambda i: (i, 0)),
            pl.BlockSpec((BB * N, 2 * T), lambda i: (i, 0)),
            pl.BlockSpec((2 * T, F), lambda i: (0, 0)),
            pl.BlockSpec((F,), lambda i: (0,)),
            pl.BlockSpec((F, DH), lambda i: (0, 0)),
            pl.BlockSpec((DH,), lambda i: (0,)),
            pl.BlockSpec((DH, DH), lambda i: (0, 0)),
            pl.BlockSpec((DH,), lambda i: (0,)),
            pl.BlockSpec((DH, DH), lambda i: (0, 0)),
            pl.BlockSpec((DH,), lambda i: (0,)),
        ],
        out_specs=[
            pl.BlockSpec((BB * N, DH), lambda i: (i, 0)),
            pl.BlockSpec((BB * N,), lambda i: (i,)),
        ],
        out_shape=[
            jax.ShapeDtypeStruct((B * N, DH), jnp.float32),
            jax.ShapeDtypeStruct((B * N,), jnp.float32),
        ],
        interpret=interpret,
    )(xe_flat, xn_flat, W_exp, b_tile, W1, b1, W2, b2, W3, b3)


def _segment_part(x_nei_2d, f_re, mf, Wce, bce):
    p_nei = x_nei_2d[..., -1, :]
    f_distance = jnp.sqrt(jnp.sum(p_nei ** 2, axis=-1))
    f_angle = jnp.arctan2(p_nei[..., 0], p_nei[..., 1])
    f_angle = jnp.mod(f_angle, 2.0 * np.pi)
    pidx = (f_angle / (2.0 * np.pi / P)).astype(jnp.int32)
    nm = mf.reshape(B, N).astype(jnp.int32)
    pidx = pidx * nm + (-1) * (1 - nm)
    f_re_r = f_re.reshape(B, N, DH)
    pos_list, re_list = [], []
    for p in range(P):
        m = (pidx == p).astype(jnp.float32)
        n = jnp.sum(m, axis=-1) + 0.0001
        d_mean = jnp.sum(f_distance * m, axis=-1) / n
        a_mean = jnp.sum(f_angle * m, axis=-1) / n
        pos_list.append(jnp.stack([d_mean, a_mean], axis=-1))
        re_list.append(jnp.sum(f_re_r * m[..., None], axis=-2) / n[..., None])
    positions = jnp.stack(pos_list, axis=-2)
    re_partitions = jnp.stack(re_list, axis=-2)
    f_pos = jax.nn.relu(positions @ Wce + bce)
    return jnp.concatenate([re_partitions, f_pos], axis=-1)


def kernel(x_ego_2d, x_nei_2d, W_tre, b_tre, W1, b1, W2, b2, W3, b3, Wce, bce):
    xe_flat = x_ego_2d.reshape(B, 2 * T)
    xn_flat = x_nei_2d.reshape(B * N, 2 * T)
    W_exp = (jnp.asarray(_M0)[:, :, None] * W_tre[0][None, None, :]
             + jnp.asarray(_M1)[:, :, None] * W_tre[1][None, None, :]
             ).reshape(2 * T, F)
    b_tile = jnp.tile(b_tre, T)
    f_re, mf = _tc_mlp(xe_flat, xn_flat, W_exp, b_tile, W1, b1, W2, b2,
                       W3, b3)
    re_matrix = _segment_part(x_nei_2d, f_re, mf, Wce, bce)
    return (re_matrix, f_re.reshape(B, N, DH))
